# baseline (speedup 1.0000x reference)
import functools

import jax
import jax.numpy as jnp
from jax import lax
from jax.experimental import pallas as pl
from jax.experimental.pallas import tpu as pltpu
from jax.experimental.pallas import tpu_sc as plsc

_BATCH = 16384
_NUM_EMB = 1000000
_HIST = 50
_DIM = 64
_LANES = 16

_info = plsc.get_sparse_core_info()
_NC, _NS = _info.num_cores, _info.num_subcores
_NW = _NC * _NS
_NGROUP = _BATCH // 128
_GPW = _NGROUP // _NW

_mesh = plsc.VectorSubcoreMesh(core_axis_name="c", subcore_axis_name="s")


@functools.partial(
    pl.kernel,
    mesh=_mesh,
    out_type=jax.ShapeDtypeStruct((_HIST, 8, 128, 8, 128), jnp.float32),
    scratch_types=[
        pltpu.VMEM((128, _HIST), jnp.int32),
        pltpu.VMEM((_HIST, 128), jnp.int32),
        pltpu.VMEM((_HIST, 144), jnp.int32),
        pltpu.VMEM((128, 2 * _DIM), jnp.float32),
        pltpu.VMEM((128, 2 * _DIM), jnp.float32),
        pltpu.VMEM((8, 8, 129), jnp.float32),
        pltpu.VMEM((8, 8, 129), jnp.float32),
        pltpu.SemaphoreType.DMA((2,)),
        pltpu.SemaphoreType.DMA((2,)),
    ],
    compiler_params=pltpu.CompilerParams(
        use_tc_tiling_on_sc=False, needs_layout_passes=False
    ),
)
def _gather_kernel(table_hbm, idx_hbm, out_hbm,
                   idx_v, idxT, offT, rows0, rows1, tbuf0, tbuf1, gsem, ssem):
    wid = lax.axis_index("s") * _NC + lax.axis_index("c")
    iota = lax.iota(jnp.int32, _LANES)
    rowvecs = [iota + _LANES * j for j in range(8)]
    t2vecs = [2 * m + (iota >> 3) for m in range(4)]
    svec = iota & 7
    rows = (rows0, rows1)
    tbufs = (tbuf0, tbuf1)

    def fire(h, q):
        pltpu.async_copy(table_hbm.at[idxT.at[h]], rows[q], gsem.at[q])

    def wait_gather(h, q):
        pltpu.make_async_copy(
            table_hbm.at[idxT.at[h]], rows[q], gsem.at[q]
        ).wait()

    def wait_store(g, h, q):
        pltpu.make_async_copy(
            tbufs[q].at[:, :, pl.ds(0, 128)], out_hbm.at[h, :, g], ssem.at[q]
        ).wait()

    def transpose_tiles(q, h):
        rbuf = rows[q]
        tbuf = tbufs[q]
        @plsc.parallel_loop(0, 128, unroll=6)
        def _l_loop(l):
            lv = jnp.full((_LANES,), l, jnp.int32)
            off = offT[h, pl.ds(l, _LANES)][0]
            for m in range(4):
                v = rbuf[l, pl.ds(off + _LANES * m, _LANES)]
                plsc.store_scatter(tbuf, [t2vecs[m], svec, lv], v)

    for k in range(_GPW):
        g = wid * _GPW + k
        b0 = g * 128
        pltpu.sync_copy(idx_hbm.at[pl.ds(b0, 128)], idx_v)

        @plsc.parallel_loop(0, _HIST, unroll=5)
        def _tr_idx(h):
            col = jnp.full((_LANES,), h, jnp.int32)
            for m in range(8):
                v = plsc.load_gather(idx_v, [rowvecs[m], col])
                idxT[h, pl.ds(_LANES * m, _LANES)] = v >> 1
                offT[h, pl.ds(_LANES * m, _LANES)] = (v & 1) << 6

        fire(0, 0)
        def hbody(i, carry):
            for q in (0, 1):
                h = 2 * i + q
                @pl.when(h + 1 < _HIST)
                def _prefetch():
                    fire(h + 1, 1 - q)
                wait_gather(h, q)
                if k == 0:
                    @pl.when(h >= 2)
                    def _reuse():
                        wait_store(g, h, q)
                else:
                    wait_store(g, h, q)
                transpose_tiles(q, h)
                pltpu.async_copy(tbufs[q].at[:, :, pl.ds(0, 128)], out_hbm.at[h, :, g], ssem.at[q])
            return carry
        lax.fori_loop(0, _HIST // 2, hbody, 0)

    g_last = wid * _GPW + _GPW - 1
    wait_store(g_last, _HIST - 2, 0)
    wait_store(g_last, _HIST - 1, 1)


def kernel(input_ids, weight):
    p = _gather_kernel(weight.reshape(_NUM_EMB // 2, 2 * _DIM), input_ids)
    return p.transpose(2, 4, 0, 1, 3).reshape(_BATCH, _HIST, _DIM)


# speedup vs baseline: 1.0899x; 1.0899x over previous
import functools

import jax
import jax.numpy as jnp
from jax import lax
from jax.experimental import pallas as pl
from jax.experimental.pallas import tpu as pltpu
from jax.experimental.pallas import tpu_sc as plsc

_BATCH = 16384
_HIST = 50
_DIM = 64
_LANES = 16

_info = plsc.get_sparse_core_info()
_NC, _NS = _info.num_cores, _info.num_subcores
_NW = _NC * _NS
_NGROUP = _BATCH // 128
_GPW = _NGROUP // _NW

_mesh = plsc.VectorSubcoreMesh(core_axis_name="c", subcore_axis_name="s")


@functools.partial(
    pl.kernel,
    mesh=_mesh,
    out_type=jax.ShapeDtypeStruct((_HIST, 8, 128, 8, 128), jnp.float32),
    scratch_types=[
        pltpu.VMEM((128, _HIST), jnp.int32),
        pltpu.VMEM((_HIST, 128), jnp.int32),
        pltpu.VMEM((128, _DIM), jnp.float32),
        pltpu.VMEM((128, _DIM), jnp.float32),
        pltpu.VMEM((8, 8, 129), jnp.float32),
        pltpu.VMEM((8, 8, 129), jnp.float32),
        pltpu.SemaphoreType.DMA((2,)),
        pltpu.SemaphoreType.DMA((2,)),
    ],
    compiler_params=pltpu.CompilerParams(
        use_tc_tiling_on_sc=False, needs_layout_passes=False
    ),
)
def _gather_kernel(table_hbm, idx_hbm, out_hbm,
                   idx_v, idxT, rows0, rows1, tbuf0, tbuf1, gsem, ssem):
    wid = lax.axis_index("s") * _NC + lax.axis_index("c")
    iota = lax.iota(jnp.int32, _LANES)
    rowvecs = [iota + _LANES * j for j in range(8)]
    t2vecs = [2 * m + (iota >> 3) for m in range(4)]
    svec = iota & 7
    rows = (rows0, rows1)
    tbufs = (tbuf0, tbuf1)

    def fire(h, q):
        pltpu.async_copy(table_hbm.at[idxT.at[h]], rows[q], gsem.at[q])

    def wait_gather(h, q):
        pltpu.make_async_copy(
            table_hbm.at[idxT.at[h]], rows[q], gsem.at[q]
        ).wait()

    def wait_store(g, h, q):
        pltpu.make_async_copy(
            tbufs[q].at[:, :, pl.ds(0, 128)], out_hbm.at[h, :, g], ssem.at[q]
        ).wait()

    def transpose_tiles(q):
        rbuf = rows[q]
        tbuf = tbufs[q]
        @plsc.parallel_loop(0, 128, unroll=6)
        def _l_loop(l):
            lv = jnp.full((_LANES,), l, jnp.int32)
            for m in range(4):
                v = rbuf[l, pl.ds(_LANES * m, _LANES)]
                plsc.store_scatter(tbuf, [t2vecs[m], svec, lv], v)

    for k in range(_GPW):
        g = wid * _GPW + k
        b0 = g * 128
        pltpu.sync_copy(idx_hbm.at[pl.ds(b0, 128)], idx_v)

        @plsc.parallel_loop(0, _HIST, unroll=5)
        def _tr_idx(h):
            col = jnp.full((_LANES,), h, jnp.int32)
            for m in range(8):
                v = plsc.load_gather(idx_v, [rowvecs[m], col])
                idxT[h, pl.ds(_LANES * m, _LANES)] = v

        fire(0, 0)
        def hbody(i, carry):
            for q in (0, 1):
                h = 2 * i + q
                @pl.when(h + 1 < _HIST)
                def _prefetch():
                    fire(h + 1, 1 - q)
                wait_gather(h, q)
                if k == 0:
                    @pl.when(h >= 2)
                    def _reuse():
                        wait_store(g, h, q)
                else:
                    wait_store(g, h, q)
                transpose_tiles(q)
                pltpu.async_copy(tbufs[q].at[:, :, pl.ds(0, 128)], out_hbm.at[h, :, g], ssem.at[q])
            return carry
        lax.fori_loop(0, _HIST // 2, hbody, 0)

    g_last = wid * _GPW + _GPW - 1
    wait_store(g_last, _HIST - 2, 0)
    wait_store(g_last, _HIST - 1, 1)


def kernel(input_ids, weight):
    p = _gather_kernel(weight, input_ids)
    return p.transpose(2, 4, 0, 1, 3).reshape(_BATCH, _HIST, _DIM)
